# baseline (speedup 1.0000x reference)
import functools

import jax
import jax.numpy as jnp
from jax import lax
from jax.experimental import pallas as pl
from jax.experimental.pallas import tpu as pltpu
from jax.experimental.pallas import tpu_sc as plsc

VB = 4096
BW = 128


def _tc_transpose_kernel(in_ref, out_ref):
    t = in_ref[...]
    out_ref[:, 0:64] = t[:, 0:VB // 2].T
    out_ref[:, 64:128] = t[:, VB // 2:VB].T


def _sc_gather_kernel(xt_hbm, tab_hbm, out_hbm, idx_v, idx_p, G, GT2, gsems,
                      osems, *, s_len, d):
    wid = lax.axis_index("s") * 2 + lax.axis_index("c")
    b0 = wid * BW
    pltpu.sync_copy(xt_hbm.at[:, pl.ds(b0, BW)], idx_v)

    def pre_body(s, _):
        for bq in range(0, BW, 16):
            kv = idx_v[s, pl.ds(bq, 16)]
            idx_p[s, pl.ds(bq, 16)] = lax.shift_right_logical(kv, 1)
        return 0
    lax.fori_loop(0, s_len, pre_body, 0)

    def fire_gather(s, p):
        pltpu.async_copy(tab_hbm.at[idx_p.at[s]], G.at[p], gsems.at[p])

    def wait_gather(p):
        pltpu.make_async_copy(tab_hbm.at[pl.ds(0, BW)], G.at[p],
                              gsems.at[p]).wait()

    def fire_copy(s, p):
        pltpu.async_copy(GT2.at[p].at[:, pl.ds(0, BW)],
                         out_hbm.at[s, :, pl.ds(b0, BW)], osems.at[p])

    def wait_copy(p):
        pltpu.make_async_copy(GT2.at[p].at[:, pl.ds(0, BW)],
                              out_hbm.at[0, :, pl.ds(b0, BW)],
                              osems.at[p]).wait()

    eidx = [lax.iota(jnp.int32, 16) + e0 for e0 in range(0, d, 16)]

    def transpose_tile(s, p):
        def group_body(g, _):
            bq = g * 16
            hv = (idx_v[s, pl.ds(bq, 16)] & 1) * d
            for l in range(16):
                h = hv[l]
                b = bq + l
                bv = jnp.zeros((16,), jnp.int32) + b
                for j, e0 in enumerate(range(0, d, 16)):
                    v = G.at[p][b, pl.ds(h + e0, 16)]
                    plsc.store_scatter(GT2.at[p], [eidx[j], bv], v)
            return 0
        lax.fori_loop(0, BW // 16, group_body, 0)

    fire_gather(0, 0)

    def body(i, _):
        s = 2 * i
        wait_gather(0)
        fire_gather(s + 1, 1)

        @pl.when(i > 0)
        def _():
            wait_copy(0)
        transpose_tile(s, 0)
        fire_copy(s, 0)

        wait_gather(1)

        @pl.when(i < s_len // 2 - 1)
        def _():
            fire_gather(s + 2, 0)

        @pl.when(i > 0)
        def _():
            wait_copy(1)
        transpose_tile(s + 1, 1)
        fire_copy(s + 1, 1)
        return 0

    lax.fori_loop(0, s_len // 2, body, 0)
    wait_copy(0)
    wait_copy(1)


def kernel(x, table):
    b, s = x.shape
    v, d = table.shape
    nblk = (v + VB - 1) // VB
    v2 = nblk * VB

    xi = x.astype(jnp.int32)
    xk = (xi - (xi & (VB - 1)) + 2 * (xi & (VB // 2 - 1))
          + ((xi >> 11) & 1))
    xt = xk.T

    tab_t = table.T

    tab_pair = pl.pallas_call(
        _tc_transpose_kernel,
        grid=(nblk,),
        in_specs=[pl.BlockSpec((d, VB), lambda i: (0, i))],
        out_specs=pl.BlockSpec((VB // 2, 2 * d), lambda i: (i, 0)),
        out_shape=jax.ShapeDtypeStruct((v2 // 2, 2 * d), jnp.float32),
    )(tab_t)

    mesh = plsc.VectorSubcoreMesh(core_axis_name="c", subcore_axis_name="s")
    k = functools.partial(
        pl.kernel,
        mesh=mesh,
        compiler_params=pltpu.CompilerParams(
            use_tc_tiling_on_sc=True, needs_layout_passes=False),
        out_type=jax.ShapeDtypeStruct((s, d, b), jnp.float32),
        scratch_types=[
            pltpu.VMEM((s, BW), jnp.int32),
            pltpu.VMEM((s, BW), jnp.int32),
            pltpu.VMEM((2, BW, 2 * d), jnp.float32),
            pltpu.VMEM((2, d, BW + 1), jnp.float32),
            pltpu.SemaphoreType.DMA((2,)),
            pltpu.SemaphoreType.DMA((2,)),
        ],
    )(functools.partial(_sc_gather_kernel, s_len=s, d=d))

    out_t = k(xt, tab_pair)
    return lax.transpose(out_t, (2, 0, 1))


# speedup vs baseline: 1.8949x; 1.8949x over previous
import functools

import jax
import jax.numpy as jnp
from jax import lax
from jax.experimental import pallas as pl
from jax.experimental.pallas import tpu as pltpu
from jax.experimental.pallas import tpu_sc as plsc

VB = 8192
BW = 128


def _tc_transpose_kernel(in_ref, out_ref):
    t = in_ref[...]
    out_ref[:, 0:64] = t[:, 0:VB // 2].T
    out_ref[:, 64:128] = t[:, VB // 2:VB].T


def _sc_gather_kernel(xt_hbm, tab_hbm, out_hbm, idx_v, G, GT2, gsems,
                      osems, *, s_len, d):
    wid = lax.axis_index("s") * 2 + lax.axis_index("c")
    b0 = wid * BW
    pltpu.sync_copy(xt_hbm.at[:, pl.ds(b0, BW)], idx_v)

    def fire_gather(s, p):
        pltpu.async_copy(tab_hbm.at[idx_v.at[s]], G.at[p], gsems.at[p])

    def wait_gather(p):
        pltpu.make_async_copy(tab_hbm.at[pl.ds(0, BW)], G.at[p],
                              gsems.at[p]).wait()

    def fire_copy(s, p):
        pltpu.async_copy(GT2.at[p].at[:, pl.ds(0, BW)],
                         out_hbm.at[s, :, pl.ds(b0, BW)], osems.at[p])

    def wait_copy(p):
        pltpu.make_async_copy(GT2.at[p].at[:, pl.ds(0, BW)],
                              out_hbm.at[0, :, pl.ds(b0, BW)],
                              osems.at[p]).wait()

    eidx = [lax.iota(jnp.int32, 16) + e0 for e0 in range(0, d, 16)]

    def transpose_tile(p):
        @plsc.parallel_loop(0, BW, 1, unroll=8)
        def _(b):
            bv = jnp.zeros((16,), jnp.int32) + b
            for j, e0 in enumerate(range(0, d, 16)):
                v = G.at[p][b, pl.ds(e0, 16)]
                plsc.store_scatter(GT2.at[p], [eidx[j], bv], v)

    fire_gather(0, 0)

    def body(i, _):
        s = 2 * i
        wait_gather(0)
        fire_gather(s + 1, 1)

        @pl.when(i > 0)
        def _():
            wait_copy(0)
        transpose_tile(0)
        fire_copy(s, 0)

        wait_gather(1)

        @pl.when(i < s_len // 2 - 1)
        def _():
            fire_gather(s + 2, 0)

        @pl.when(i > 0)
        def _():
            wait_copy(1)
        transpose_tile(1)
        fire_copy(s + 1, 1)
        return 0

    lax.fori_loop(0, s_len // 2, body, 0)
    wait_copy(0)
    wait_copy(1)


def kernel(x, table):
    b, s = x.shape
    v, d = table.shape
    nblk = (v + VB - 1) // VB
    v2 = nblk * VB

    xi = x.astype(jnp.int32)
    xk = (xi - (xi & (VB - 1)) + 2 * (xi & (VB // 2 - 1))
          + ((xi >> 12) & 1))
    xt = xk.T

    tab_t = table.T

    tab_pair = pl.pallas_call(
        _tc_transpose_kernel,
        grid=(nblk,),
        in_specs=[pl.BlockSpec((d, VB), lambda i: (0, i))],
        out_specs=pl.BlockSpec((VB // 2, 2 * d), lambda i: (i, 0)),
        out_shape=jax.ShapeDtypeStruct((v2 // 2, 2 * d), jnp.float32),
    )(tab_t)
    tab_rm = tab_pair.reshape(v2, d)

    mesh = plsc.VectorSubcoreMesh(core_axis_name="c", subcore_axis_name="s")
    k = functools.partial(
        pl.kernel,
        mesh=mesh,
        compiler_params=pltpu.CompilerParams(
            use_tc_tiling_on_sc=False, needs_layout_passes=False),
        out_type=jax.ShapeDtypeStruct((s, d, b), jnp.float32),
        scratch_types=[
            pltpu.VMEM((s, BW), jnp.int32),
            pltpu.VMEM((2, BW, d), jnp.float32),
            pltpu.VMEM((2, d, BW + 1), jnp.float32),
            pltpu.SemaphoreType.DMA((2,)),
            pltpu.SemaphoreType.DMA((2,)),
        ],
    )(functools.partial(_sc_gather_kernel, s_len=s, d=d))

    out_t = k(xt, tab_rm)
    return lax.transpose(out_t, (2, 0, 1))


# speedup vs baseline: 1.9820x; 1.0460x over previous
import functools

import jax
import jax.numpy as jnp
from jax import lax
from jax.experimental import pallas as pl
from jax.experimental.pallas import tpu as pltpu
from jax.experimental.pallas import tpu_sc as plsc

VB = 16384
BW = 128


def _tc_transpose_kernel(in_ref, out_ref):
    t = in_ref[...]
    out_ref[:, 0:64] = t[:, 0:VB // 2].T
    out_ref[:, 64:128] = t[:, VB // 2:VB].T


def _sc_gather_kernel(xt_hbm, tab_hbm, out_hbm, idx_v, G, GT2, gsems,
                      osems, *, s_len, d):
    wid = lax.axis_index("s") * 2 + lax.axis_index("c")
    b0 = wid * BW
    pltpu.sync_copy(xt_hbm.at[:, pl.ds(b0, BW)], idx_v)

    def fire_gather(s, p):
        pltpu.async_copy(tab_hbm.at[idx_v.at[s]], G.at[p], gsems.at[p])

    def wait_gather(p):
        pltpu.make_async_copy(tab_hbm.at[pl.ds(0, BW)], G.at[p],
                              gsems.at[p]).wait()

    def fire_copy(s, p):
        pltpu.async_copy(GT2.at[p].at[:, pl.ds(0, BW)],
                         out_hbm.at[s, :, pl.ds(b0, BW)], osems.at[p])

    def wait_copy(p):
        pltpu.make_async_copy(GT2.at[p].at[:, pl.ds(0, BW)],
                              out_hbm.at[0, :, pl.ds(b0, BW)],
                              osems.at[p]).wait()

    eidx = [lax.iota(jnp.int32, 16) + e0 for e0 in range(0, d, 16)]

    def transpose_tile(p):
        @plsc.parallel_loop(0, BW, 1, unroll=8)
        def _(b):
            bv = jnp.zeros((16,), jnp.int32) + b
            for j, e0 in enumerate(range(0, d, 16)):
                v = G.at[p][b, pl.ds(e0, 16)]
                plsc.store_scatter(GT2.at[p], [eidx[j], bv], v)

    fire_gather(0, 0)

    def body(i, _):
        s = 2 * i
        wait_gather(0)
        fire_gather(s + 1, 1)

        @pl.when(i > 0)
        def _():
            wait_copy(0)
        transpose_tile(0)
        fire_copy(s, 0)

        wait_gather(1)

        @pl.when(i < s_len // 2 - 1)
        def _():
            fire_gather(s + 2, 0)

        @pl.when(i > 0)
        def _():
            wait_copy(1)
        transpose_tile(1)
        fire_copy(s + 1, 1)
        return 0

    lax.fori_loop(0, s_len // 2, body, 0)
    wait_copy(0)
    wait_copy(1)


def kernel(x, table):
    b, s = x.shape
    v, d = table.shape
    nblk = (v + VB - 1) // VB
    v2 = nblk * VB

    xi = x.astype(jnp.int32)
    xk = (xi - (xi & (VB - 1)) + 2 * (xi & (VB // 2 - 1))
          + ((xi >> 13) & 1))
    xt = xk.T

    tab_t = table.T

    tab_pair = pl.pallas_call(
        _tc_transpose_kernel,
        grid=(nblk,),
        in_specs=[pl.BlockSpec((d, VB), lambda i: (0, i))],
        out_specs=pl.BlockSpec((VB // 2, 2 * d), lambda i: (i, 0)),
        out_shape=jax.ShapeDtypeStruct((v2 // 2, 2 * d), jnp.float32),
    )(tab_t)
    tab_rm = tab_pair.reshape(v2, d)

    mesh = plsc.VectorSubcoreMesh(core_axis_name="c", subcore_axis_name="s")
    k = functools.partial(
        pl.kernel,
        mesh=mesh,
        compiler_params=pltpu.CompilerParams(
            use_tc_tiling_on_sc=False, needs_layout_passes=False),
        out_type=jax.ShapeDtypeStruct((s, d, b), jnp.float32),
        scratch_types=[
            pltpu.VMEM((s, BW), jnp.int32),
            pltpu.VMEM((2, BW, d), jnp.float32),
            pltpu.VMEM((2, d, BW + 1), jnp.float32),
            pltpu.SemaphoreType.DMA((2,)),
            pltpu.SemaphoreType.DMA((2,)),
        ],
    )(functools.partial(_sc_gather_kernel, s_len=s, d=d))

    out_t = k(xt, tab_rm)
    return lax.transpose(out_t, (2, 0, 1))


# speedup vs baseline: 2.0202x; 1.0193x over previous
import functools

import jax
import jax.numpy as jnp
from jax import lax
from jax.experimental import pallas as pl
from jax.experimental.pallas import tpu as pltpu
from jax.experimental.pallas import tpu_sc as plsc

VB = 32768
BW = 128


def _tc_transpose_kernel(in_ref, out_ref):
    t = in_ref[...]
    out_ref[:, 0:64] = t[:, 0:VB // 2].T
    out_ref[:, 64:128] = t[:, VB // 2:VB].T


def _sc_gather_kernel(xt_hbm, tab_hbm, out_hbm, idx_v, G, GT2, gsems,
                      osems, *, s_len, d):
    wid = lax.axis_index("s") * 2 + lax.axis_index("c")
    b0 = wid * BW
    pltpu.sync_copy(xt_hbm.at[:, pl.ds(b0, BW)], idx_v)

    def fire_gather(s, p):
        pltpu.async_copy(tab_hbm.at[idx_v.at[s]], G.at[p], gsems.at[p])

    def wait_gather(p):
        pltpu.make_async_copy(tab_hbm.at[pl.ds(0, BW)], G.at[p],
                              gsems.at[p]).wait()

    def fire_copy(s, p):
        pltpu.async_copy(GT2.at[p].at[:, pl.ds(0, BW)],
                         out_hbm.at[s, :, pl.ds(b0, BW)], osems.at[p])

    def wait_copy(p):
        pltpu.make_async_copy(GT2.at[p].at[:, pl.ds(0, BW)],
                              out_hbm.at[0, :, pl.ds(b0, BW)],
                              osems.at[p]).wait()

    eidx = [lax.iota(jnp.int32, 16) + e0 for e0 in range(0, d, 16)]

    def transpose_tile(p):
        @plsc.parallel_loop(0, BW, 1, unroll=8)
        def _(b):
            bv = jnp.zeros((16,), jnp.int32) + b
            for j, e0 in enumerate(range(0, d, 16)):
                v = G.at[p][b, pl.ds(e0, 16)]
                plsc.store_scatter(GT2.at[p], [eidx[j], bv], v)

    fire_gather(0, 0)

    def body(i, _):
        s = 2 * i
        wait_gather(0)
        fire_gather(s + 1, 1)

        @pl.when(i > 0)
        def _():
            wait_copy(0)
        transpose_tile(0)
        fire_copy(s, 0)

        wait_gather(1)

        @pl.when(i < s_len // 2 - 1)
        def _():
            fire_gather(s + 2, 0)

        @pl.when(i > 0)
        def _():
            wait_copy(1)
        transpose_tile(1)
        fire_copy(s + 1, 1)
        return 0

    lax.fori_loop(0, s_len // 2, body, 0)
    wait_copy(0)
    wait_copy(1)


def kernel(x, table):
    b, s = x.shape
    v, d = table.shape
    nblk = (v + VB - 1) // VB
    v2 = nblk * VB

    xi = x.astype(jnp.int32)
    xk = (xi - (xi & (VB - 1)) + 2 * (xi & (VB // 2 - 1))
          + ((xi >> 14) & 1))
    xt = xk.T

    tab_t = table.T

    tab_pair = pl.pallas_call(
        _tc_transpose_kernel,
        grid=(nblk,),
        in_specs=[pl.BlockSpec((d, VB), lambda i: (0, i))],
        out_specs=pl.BlockSpec((VB // 2, 2 * d), lambda i: (i, 0)),
        out_shape=jax.ShapeDtypeStruct((v2 // 2, 2 * d), jnp.float32),
    )(tab_t)
    tab_rm = tab_pair.reshape(v2, d)

    mesh = plsc.VectorSubcoreMesh(core_axis_name="c", subcore_axis_name="s")
    k = functools.partial(
        pl.kernel,
        mesh=mesh,
        compiler_params=pltpu.CompilerParams(
            use_tc_tiling_on_sc=False, needs_layout_passes=False),
        out_type=jax.ShapeDtypeStruct((s, d, b), jnp.float32),
        scratch_types=[
            pltpu.VMEM((s, BW), jnp.int32),
            pltpu.VMEM((2, BW, d), jnp.float32),
            pltpu.VMEM((2, d, BW + 1), jnp.float32),
            pltpu.SemaphoreType.DMA((2,)),
            pltpu.SemaphoreType.DMA((2,)),
        ],
    )(functools.partial(_sc_gather_kernel, s_len=s, d=d))

    out_t = k(xt, tab_rm)
    return lax.transpose(out_t, (2, 0, 1))
